# baseline (speedup 1.0000x reference)
import math

import numpy as np

import jax
import jax.numpy as jnp
from jax.experimental import pallas as pl
from jax.experimental.pallas import tpu as pltpu

S = 2048
D = 1024
KC = 5
F_BR = 4
BD = 128
LB = 128
TK = 8
PC = 64
NEG = -1e30
BLK = 256
NBLK = S // BLK
WIN = BLK + LB
SCALE = 1.0 / math.sqrt(D)

_r = np.arange(BLK)[:, None]
_c = np.arange(BLK)[None, :]
_AMAT = np.where((_r // PC == _c // PC) & (_c <= _r),
                 1.0 / (_r % PC + 1), 0.0).astype(np.float32)
_cw = np.arange(WIN)[None, :]
_BANDM = ((_cw >= _r) & (_cw <= _r + LB - 1)).astype(np.int8)


def _mm_t(x, w):
    return jax.lax.dot_general(x, w, (((1,), (1,)), ((), ())),
                               preferred_element_type=jnp.float32)


def _body(h_ref, Wq_hbm, Wk_hbm, Wv_hbm, Wpk_hbm, Wpv_hbm, Wprop_hbm,
          Wback_hbm, Wct_ref, bconv_ref, bq_ref, bk_ref, bv_ref,
          bpk_ref, bpv_ref, bprop_ref, bback_ref,
          Wsc_ref, Wg_ref, Amat_ref, bandm_ref,
          bscore_ref, bgate_ref,
          out_ref, Wq_s, Wk_s, Wv_s, Wpk_s, Wpv_s, Wprop_s, Wback_s,
          stage0, stage1, stageb, sem0, sem1, semb,
          ktail, vtail, pklast, pvlast, htail):
    i = pl.program_id(0)
    start = i * BLK
    bf = jnp.bfloat16

    @pl.when(i == 0)
    def _load_weights():
        hbm = [Wq_hbm, Wk_hbm, Wv_hbm, Wpk_hbm, Wpv_hbm]
        dst = [Wq_s, Wk_s, Wv_s, Wpk_s, Wpv_s]
        stages = [stage0, stage1]
        sems = [sem0, sem1]
        cps = [pltpu.make_async_copy(hbm[n], stages[n % 2], sems[n % 2])
               for n in range(5)]
        cpp = pltpu.make_async_copy(Wprop_hbm, stage0.at[0:F_BR * BD, :],
                                    sem0)
        cpb = pltpu.make_async_copy(Wback_hbm, stageb, semb)
        cps[0].start()
        cps[1].start()
        cpb.start()
        for n in range(5):
            cps[n].wait()
            dst[n][...] = stages[n % 2][...].astype(bf)
            if n + 2 < 5:
                cps[n + 2].start()
        cpp.start()
        cpp.wait()
        Wprop_s[...] = stage0[0:F_BR * BD, :].astype(bf)
        cpb.wait()
        Wback_s[...] = stageb[...].astype(bf)

    h_blk = h_ref[...]
    halo = jnp.where(i == 0, jnp.zeros_like(htail[...]), htail[...])
    hx = jnp.concatenate([halo, h_blk], axis=0).astype(bf)
    Wct = Wct_ref[...].T.astype(bf)
    conv = hx[0:BLK, :] * Wct[0:1, :]
    for j in range(1, KC):
        conv = conv + hx[j:j + BLK, :] * Wct[j:j + 1, :]
    h1 = h_blk + conv.astype(jnp.float32) + bconv_ref[...]
    htail[...] = h_blk[BLK - (KC - 1):, :]

    ph = jnp.dot(Amat_ref[...], h1, preferred_element_type=jnp.float32)

    h1b = h1.astype(bf)
    phb = ph.astype(bf)
    q = _mm_t(h1b, Wq_s[...]) + bq_ref[...]
    k = _mm_t(h1b, Wk_s[...]) + bk_ref[...]
    v = _mm_t(h1b, Wv_s[...]) + bv_ref[...]
    pk = _mm_t(phb, Wpk_s[...]) + bpk_ref[...]
    pv = _mm_t(phb, Wpv_s[...]) + bpv_ref[...]
    p = _mm_t(h1b, Wprop_s[...]) + bprop_ref[...]

    kb = k.astype(bf)
    vb = v.astype(bf)
    kt = jnp.where(i == 0, jnp.zeros_like(ktail[...]), ktail[...])
    vt = jnp.where(i == 0, jnp.zeros_like(vtail[...]), vtail[...])
    k_win = jnp.concatenate([kt, kb], axis=0)
    v_win = jnp.concatenate([vt, vb], axis=0)
    qb = q.astype(bf)
    scores_f = _mm_t(qb, k_win) * SCALE

    colid = jax.lax.broadcasted_iota(jnp.int32, (BLK, WIN), 1)
    okpos = (i > 0) | (colid >= LB)
    validm = (bandm_ref[...] != 0) & okpos
    scores_f = jnp.where(validm, scores_f, NEG)
    scores = scores_f.astype(bf)

    rowid = jax.lax.broadcasted_iota(jnp.int32, (BLK, WIN), 0)
    seq_s = jnp.sum(jnp.where(colid == rowid + LB - 1, scores_f, 0.0),
                    axis=1, keepdims=True)

    pk_prev = jnp.concatenate(
        [jnp.where(i == 0, pk[0:1, :], pklast[...]), pk[:BLK - 1, :]], axis=0)
    pv_prev = jnp.concatenate(
        [jnp.where(i == 0, pv[0:1, :], pvlast[...]), pv[:BLK - 1, :]], axis=0)
    v_last = jnp.where(i == 0, v[0:1, :],
                       vtail[LB - 1:LB, :].astype(jnp.float32))
    v_prev = jnp.concatenate([v_last, v[:BLK - 1, :]], axis=0)

    ph_s = jnp.sum(q * pk_prev, axis=1, keepdims=True) * SCALE

    m0 = jnp.max(scores, axis=1, keepdims=True)
    M = jnp.maximum(jnp.maximum(m0.astype(jnp.float32), seq_s), ph_s)
    eduo = jnp.exp(jnp.concatenate([seq_s, ph_s], axis=1) - M)
    e_seq = eduo[:, 0:1]
    e_ph = eduo[:, 1:2]
    minf = jnp.asarray(-jnp.inf, bf)
    cur = scores
    m = m0
    for it in range(TK - 1):
        cur = jnp.where(cur == m, minf, cur)
        m = jnp.max(cur, axis=1, keepdims=True)
    Mb = M.astype(bf)
    wnum = jnp.exp(jnp.where(scores >= m, scores - Mb, minf))
    den = (e_seq + e_ph
           + jnp.sum(wnum, axis=1, keepdims=True, dtype=jnp.float32))

    sem_part = jax.lax.dot_general(wnum, v_win,
                                   (((1,), (0,)), ((), ())),
                                   preferred_element_type=jnp.float32)
    summ = (sem_part + e_seq * v_prev + e_ph * pv_prev) / den
    trow = start + jax.lax.broadcasted_iota(jnp.int32, (BLK, 1), 0)
    summ = jnp.where(trow >= 1, summ, 0.0)

    G = jnp.concatenate([Wsc_ref[0:1, :], Wg_ref[0:1, :]], axis=0).astype(bf)
    gdots = _mm_t(h1b, G)
    h1s = gdots[:, 0:1]
    gl1 = gdots[:, 1:2]
    ws2 = Wsc_ref[1:2, :]
    u = jax.lax.dot_general(ws2.astype(bf), Wback_s[...],
                            (((1,), (0,)), ((), ())),
                            preferred_element_type=jnp.float32)
    csc = (jnp.sum(bback_ref[...] * ws2, axis=1, keepdims=True)
           + bscore_ref[...])
    pfs = [p[:, f * BD:(f + 1) * BD] for f in range(F_BR)]
    sc4 = jnp.concatenate(
        [jnp.sum(pf * u, axis=1, keepdims=True) for pf in pfs], axis=1)
    sc4 = sc4 + h1s + csc
    mx = jnp.max(sc4, axis=1, keepdims=True)
    E4 = jnp.exp(sc4 - mx)
    sume = jnp.sum(E4, axis=1, keepdims=True)
    pmix = (E4[:, 0:1] * pfs[0] + E4[:, 1:2] * pfs[1]
            + E4[:, 2:3] * pfs[2] + E4[:, 3:4] * pfs[3]) / sume
    branch = _mm_t(pmix.astype(bf), Wback_s[...]) + bback_ref[...]
    bs = branch + summ

    gl = gl1 + jnp.sum(bs * Wg_ref[1:2, :], axis=1, keepdims=True) \
        + bgate_ref[...]
    g = jax.nn.sigmoid(gl)
    out_ref[...] = h1 + g * (bs - h1)

    ktail[...] = kb[BLK - LB:, :]
    vtail[...] = vb[BLK - LB:, :]
    pklast[...] = pk[BLK - 1:, :]
    pvlast[...] = pv[BLK - 1:, :]


def kernel(h, Wconv, bconv, Wq, bq, Wk, bk, Wv, bv, Wpk, bpk, Wpv, bpv,
           Wprop, bprop, Wback, bback, Wscore, bscore, Wgate, bgate):
    f32 = jnp.float32
    bf = jnp.bfloat16
    h2 = h.reshape(S, D)

    anyspec = pl.BlockSpec(memory_space=pl.ANY)
    full = lambda s: pl.BlockSpec(s, lambda i: (0, 0))
    blocked = pl.BlockSpec((BLK, D), lambda i: (i, 0))

    out = pl.pallas_call(
        _body,
        grid=(NBLK,),
        in_specs=[
            blocked,
            anyspec, anyspec, anyspec,
            anyspec, anyspec,
            anyspec, anyspec,
            full((D, KC)),
            full((1, D)),
            full((1, D)), full((1, D)), full((1, D)),
            full((1, D)), full((1, D)),
            full((1, F_BR * BD)),
            full((1, D)),
            full((2, D)),
            full((2, D)),
            full((BLK, BLK)),
            full((BLK, WIN)),
            full((1, 1)), full((1, 1)),
        ],
        out_specs=blocked,
        out_shape=jax.ShapeDtypeStruct((S, D), f32),
        scratch_shapes=[
            pltpu.VMEM((D, D), bf),
            pltpu.VMEM((D, D), bf),
            pltpu.VMEM((D, D), bf),
            pltpu.VMEM((D, D), bf),
            pltpu.VMEM((D, D), bf),
            pltpu.VMEM((F_BR * BD, D), bf),
            pltpu.VMEM((D, BD), bf),
            pltpu.VMEM((D, D), f32),
            pltpu.VMEM((D, D), f32),
            pltpu.VMEM((D, BD), f32),
            pltpu.SemaphoreType.DMA,
            pltpu.SemaphoreType.DMA,
            pltpu.SemaphoreType.DMA,
            pltpu.VMEM((LB, D), bf),
            pltpu.VMEM((LB, D), bf),
            pltpu.VMEM((1, D), f32),
            pltpu.VMEM((1, D), f32),
            pltpu.VMEM((KC - 1, D), f32),
        ],
    )(
        h2, Wq, Wk, Wv, Wpk, Wpv, Wprop, Wback,
        Wconv, bconv.reshape(1, D),
        bq.reshape(1, D), bk.reshape(1, D), bv.reshape(1, D),
        bpk.reshape(1, D), bpv.reshape(1, D),
        bprop.reshape(1, F_BR * BD), bback.reshape(1, D),
        Wscore.reshape(2, D), Wgate.reshape(2, D),
        jnp.asarray(_AMAT), jnp.asarray(_BANDM),
        bscore.reshape(1, 1), bgate.reshape(1, 1),
    )
    return out.reshape(1, S, D)


# speedup vs baseline: 1.0950x; 1.0950x over previous
import math

import numpy as np

import jax
import jax.numpy as jnp
from jax.experimental import pallas as pl
from jax.experimental.pallas import tpu as pltpu

S = 2048
D = 1024
KC = 5
F_BR = 4
BD = 128
LB = 128
TK = 8
PC = 64
NEG = -1e30
BLK = 256
NBLK = S // BLK
WIN = BLK + LB
SCALE = 1.0 / math.sqrt(D)

_r = np.arange(BLK)[:, None]
_c = np.arange(BLK)[None, :]
_AMAT = np.where((_r // PC == _c // PC) & (_c <= _r),
                 1.0 / (_r % PC + 1), 0.0).astype(np.float32)
_cw = np.arange(WIN)[None, :]
_BANDM = ((_cw >= _r) & (_cw <= _r + LB - 1)).astype(np.int8)


def _mm_t(x, w):
    return jax.lax.dot_general(x, w, (((1,), (1,)), ((), ())),
                               preferred_element_type=jnp.float32)


def _body(h_ref, Wq_hbm, Wk_hbm, Wv_hbm, Wpk_hbm, Wpv_hbm, Wprop_hbm,
          Wback_hbm, Wct_ref, bconv_ref, bq_ref, bk_ref, bv_ref,
          bpk_ref, bpv_ref, bprop_ref, bback_ref,
          Wsc_ref, Wg_ref, Amat_ref, bandm_ref,
          bscore_ref, bgate_ref,
          out_ref, Wq_s, Wk_s, Wv_s, Wpk_s, Wpv_s, Wprop_s, Wback_s,
          stage0, stage1, stageb, sem0, sem1, semb,
          ktail, vtail, pklast, pvlast, htail):
    i = pl.program_id(0)
    start = i * BLK
    bf = jnp.bfloat16

    @pl.when(i == 0)
    def _load_weights():
        hbm = [Wq_hbm, Wk_hbm, Wv_hbm, Wpk_hbm, Wpv_hbm]
        dst = [Wq_s, Wk_s, Wv_s, Wpk_s, Wpv_s]
        stages = [stage0, stage1]
        sems = [sem0, sem1]
        cps = [pltpu.make_async_copy(hbm[n], stages[n % 2], sems[n % 2])
               for n in range(5)]
        cpp = pltpu.make_async_copy(Wprop_hbm, stage0.at[0:F_BR * BD, :],
                                    sem0)
        cpb = pltpu.make_async_copy(Wback_hbm, stageb, semb)
        cps[0].start()
        cps[1].start()
        cpb.start()
        for n in range(5):
            cps[n].wait()
            dst[n][...] = stages[n % 2][...].astype(bf)
            if n + 2 < 5:
                cps[n + 2].start()
        cpp.start()
        cpp.wait()
        Wprop_s[...] = stage0[0:F_BR * BD, :].astype(bf)
        cpb.wait()
        Wback_s[...] = stageb[...].astype(bf)

    h_blk = h_ref[...]
    halo = jnp.where(i == 0, jnp.zeros_like(htail[...]), htail[...])
    hx = jnp.concatenate([halo, h_blk], axis=0).astype(bf)
    Wct = Wct_ref[...]
    conv = hx[0:BLK, :] * Wct[0:1, :]
    for j in range(1, KC):
        conv = conv + hx[j:j + BLK, :] * Wct[j:j + 1, :]
    h1 = h_blk + conv.astype(jnp.float32) + bconv_ref[...]
    htail[...] = h_blk[BLK - (KC - 1):, :]

    ph = jnp.dot(Amat_ref[...], h1, preferred_element_type=jnp.float32)

    h1b = h1.astype(bf)
    phb = ph.astype(bf)
    q = _mm_t(h1b, Wq_s[...]) + bq_ref[...]
    k = _mm_t(h1b, Wk_s[...]) + bk_ref[...]
    v = _mm_t(h1b, Wv_s[...]) + bv_ref[...]
    pk = _mm_t(phb, Wpk_s[...]) + bpk_ref[...]
    pv = _mm_t(phb, Wpv_s[...]) + bpv_ref[...]
    p = _mm_t(h1b, Wprop_s[...]) + bprop_ref[...]

    kb = k.astype(bf)
    vb = v.astype(bf)
    kt = jnp.where(i == 0, jnp.zeros_like(ktail[...]), ktail[...])
    vt = jnp.where(i == 0, jnp.zeros_like(vtail[...]), vtail[...])
    k_win = jnp.concatenate([kt, kb], axis=0)
    v_win = jnp.concatenate([vt, vb], axis=0)
    qb = q.astype(bf)
    scores = _mm_t(qb, k_win) * SCALE

    colid = jax.lax.broadcasted_iota(jnp.int32, (BLK, WIN), 1)
    okpos = (i > 0) | (colid >= LB)
    validm = (bandm_ref[...] != 0) & okpos
    scores = jnp.where(validm, scores, NEG)

    rowid = jax.lax.broadcasted_iota(jnp.int32, (BLK, WIN), 0)
    seq_s = jnp.sum(jnp.where(colid == rowid + LB - 1, scores, 0.0),
                    axis=1, keepdims=True)

    pk_prev = jnp.concatenate(
        [jnp.where(i == 0, pk[0:1, :], pklast[...]), pk[:BLK - 1, :]], axis=0)
    pv_prev = jnp.concatenate(
        [jnp.where(i == 0, pv[0:1, :], pvlast[...]), pv[:BLK - 1, :]], axis=0)
    v_last = jnp.where(i == 0, v[0:1, :],
                       vtail[LB - 1:LB, :].astype(jnp.float32))
    v_prev = jnp.concatenate([v_last, v[:BLK - 1, :]], axis=0)

    ph_s = jnp.sum(q * pk_prev, axis=1, keepdims=True) * SCALE

    m0 = jnp.max(scores, axis=1, keepdims=True)
    M = jnp.maximum(jnp.maximum(m0, seq_s), ph_s)
    e_seq = jnp.exp(seq_s - M)
    e_ph = jnp.exp(ph_s - M)
    den = e_seq + e_ph
    wnum = jnp.zeros((BLK, WIN), jnp.float32)
    cur = scores
    minf = jnp.float32(-jnp.inf)
    for it in range(TK):
        m = m0 if it == 0 else jnp.max(cur, axis=1, keepdims=True)
        hot = cur == m
        e = jnp.exp(m - M)
        den = den + e
        wnum = wnum + jnp.where(hot, e, 0.0)
        if it < TK - 1:
            cur = jnp.where(hot, minf, cur)

    sem_part = jax.lax.dot_general(wnum.astype(bf), v_win,
                                   (((1,), (0,)), ((), ())),
                                   preferred_element_type=jnp.float32)
    summ = (sem_part + e_seq * v_prev + e_ph * pv_prev) / den
    trow = start + jax.lax.broadcasted_iota(jnp.int32, (BLK, 1), 0)
    summ = jnp.where(trow >= 1, summ, 0.0)

    G = jnp.concatenate([Wsc_ref[0:1, :], Wg_ref[0:1, :]], axis=0).astype(bf)
    gdots = _mm_t(h1b, G)
    h1s = gdots[:, 0:1]
    gl1 = gdots[:, 1:2]
    ws2 = Wsc_ref[1:2, :]
    u = jax.lax.dot_general(ws2.astype(bf), Wback_s[...],
                            (((1,), (0,)), ((), ())),
                            preferred_element_type=jnp.float32)
    csc = (jnp.sum(bback_ref[...] * ws2, axis=1, keepdims=True)
           + bscore_ref[...])
    pfs = [p[:, f * BD:(f + 1) * BD] for f in range(F_BR)]
    scs = [h1s + jnp.sum(pf * u, axis=1, keepdims=True) + csc for pf in pfs]
    mx = jnp.maximum(jnp.maximum(scs[0], scs[1]), jnp.maximum(scs[2], scs[3]))
    es = [jnp.exp(s - mx) for s in scs]
    sume = es[0] + es[1] + es[2] + es[3]
    pmix = (es[0] * pfs[0] + es[1] * pfs[1]
            + es[2] * pfs[2] + es[3] * pfs[3]) / sume
    branch = _mm_t(pmix.astype(bf), Wback_s[...]) + bback_ref[...]
    bs = branch + summ

    gl = gl1 + jnp.sum(bs * Wg_ref[1:2, :], axis=1, keepdims=True) \
        + bgate_ref[...]
    g = jax.nn.sigmoid(gl)
    out_ref[...] = h1 + g * (bs - h1)

    ktail[...] = kb[BLK - LB:, :]
    vtail[...] = vb[BLK - LB:, :]
    pklast[...] = pk[BLK - 1:, :]
    pvlast[...] = pv[BLK - 1:, :]


def kernel(h, Wconv, bconv, Wq, bq, Wk, bk, Wv, bv, Wpk, bpk, Wpv, bpv,
           Wprop, bprop, Wback, bback, Wscore, bscore, Wgate, bgate):
    f32 = jnp.float32
    bf = jnp.bfloat16
    h2 = h.reshape(S, D)

    anyspec = pl.BlockSpec(memory_space=pl.ANY)
    full = lambda s: pl.BlockSpec(s, lambda i: (0, 0))
    blocked = pl.BlockSpec((BLK, D), lambda i: (i, 0))

    out = pl.pallas_call(
        _body,
        grid=(NBLK,),
        in_specs=[
            blocked,
            anyspec, anyspec, anyspec,
            anyspec, anyspec,
            anyspec, anyspec,
            full((KC, D)),
            full((1, D)),
            full((1, D)), full((1, D)), full((1, D)),
            full((1, D)), full((1, D)),
            full((1, F_BR * BD)),
            full((1, D)),
            full((2, D)),
            full((2, D)),
            full((BLK, BLK)),
            full((BLK, WIN)),
            full((1, 1)), full((1, 1)),
        ],
        out_specs=blocked,
        out_shape=jax.ShapeDtypeStruct((S, D), f32),
        scratch_shapes=[
            pltpu.VMEM((D, D), bf),
            pltpu.VMEM((D, D), bf),
            pltpu.VMEM((D, D), bf),
            pltpu.VMEM((D, D), bf),
            pltpu.VMEM((D, D), bf),
            pltpu.VMEM((F_BR * BD, D), bf),
            pltpu.VMEM((D, BD), bf),
            pltpu.VMEM((D, D), f32),
            pltpu.VMEM((D, D), f32),
            pltpu.VMEM((D, BD), f32),
            pltpu.SemaphoreType.DMA,
            pltpu.SemaphoreType.DMA,
            pltpu.SemaphoreType.DMA,
            pltpu.VMEM((LB, D), bf),
            pltpu.VMEM((LB, D), bf),
            pltpu.VMEM((1, D), f32),
            pltpu.VMEM((1, D), f32),
            pltpu.VMEM((KC - 1, D), f32),
        ],
    )(
        h2, Wq, Wk, Wv, Wpk, Wpv, Wprop, Wback,
        Wconv.T.astype(bf), bconv.reshape(1, D),
        bq.reshape(1, D), bk.reshape(1, D), bv.reshape(1, D),
        bpk.reshape(1, D), bpv.reshape(1, D),
        bprop.reshape(1, F_BR * BD), bback.reshape(1, D),
        Wscore.reshape(2, D), Wgate.reshape(2, D),
        jnp.asarray(_AMAT), jnp.asarray(_BANDM),
        bscore.reshape(1, 1), bgate.reshape(1, 1),
    )
    return out.reshape(1, S, D)
